# baseline (speedup 1.0000x reference)
import jax
import jax.numpy as jnp
from jax.experimental import pallas as pl
from jax.experimental.pallas import tpu as pltpu

P = 256
D = 16
ED = 16


def _dot(a, b, dims):
    return jax.lax.dot_general(a, b, (dims, ((), ())),
                               preferred_element_type=jnp.float32)


def _gin_kernel(nodes_ref, nodesT_ref, We1_ref, We1T_ref, We2T_ref, wd_ref,
                be_ref, bec_ref, Wn1T_ref, Wn2T_ref, bnc_ref, out_ref, agg_scr):
    nodes = nodes_ref[0]
    nT = nodesT_ref[0]

    g = _dot(nT, nT, ((0,), (0,)))
    sq_row = jnp.sum(nT * nT, axis=0, keepdims=True)
    sq_col = jnp.sum(nodes * nodes, axis=1, keepdims=True)
    rows = jax.lax.broadcasted_iota(jnp.int32, (P, P), 0)
    cols = jax.lax.broadcasted_iota(jnp.int32, (P, P), 1)
    offdiag = (rows != cols).astype(jnp.float32)
    dist = jnp.sqrt(jnp.maximum(sq_col + sq_row - 2.0 * g, 0.0)) * offdiag

    a2 = _dot(nodes, We1_ref[...], ((1,), (0,))) + be_ref[...]
    a2T = _dot(We1T_ref[...], nT, ((1,), (0,))) + bec_ref[...]
    cT = _dot(We2T_ref[...], nT, ((1,), (0,)))

    ones_col = jnp.ones((P, 1), jnp.float32)
    ones_row = jnp.ones((1, P), jnp.float32)
    for k in range(ED):
        lhs2 = jnp.concatenate([a2[:, k:k + 1], ones_col], axis=1)
        rhs2 = jnp.concatenate([ones_row, cT[k:k + 1, :]], axis=0)
        bc = _dot(lhs2, rhs2, ((1,), (0,)))
        m = jnp.maximum(dist * wd_ref[0, k] + bc, 0.0)
        agg_scr[k:k + 1, :] = _dot(ones_row, m, ((1,), (0,)))

    aggT = agg_scr[...] - jnp.maximum(a2T + cT, 0.0)

    newT = (_dot(Wn1T_ref[...], aggT, ((1,), (0,)))
            + _dot(Wn2T_ref[...], nT, ((1,), (0,)))
            + bnc_ref[...])
    out_ref[0] = newT


def kernel(t, h, W_e, b_e, W_n, b_n):
    del t
    B = h.shape[0]
    nodes = h.reshape(B, P, D)
    nodesT = nodes.transpose(0, 2, 1)

    We1 = W_e[:D]
    We1T = We1.T
    We2T = W_e[D:2 * D].T
    wd = W_e[2 * D:2 * D + 1]
    be = b_e.reshape(1, ED)
    bec = b_e.reshape(ED, 1)
    Wn1T = W_n[:ED].T
    Wn2T = W_n[ED:].T
    bnc = b_n.reshape(D, 1)

    full = lambda shape: pl.BlockSpec(shape, lambda b: (0,) * len(shape))
    outT = pl.pallas_call(
        _gin_kernel,
        grid=(B,),
        in_specs=[
            pl.BlockSpec((1, P, D), lambda b: (b, 0, 0)),
            pl.BlockSpec((1, D, P), lambda b: (b, 0, 0)),
            full((D, ED)), full((ED, D)), full((ED, D)), full((1, ED)),
            full((1, ED)), full((ED, 1)), full((D, ED)), full((D, D)),
            full((D, 1)),
        ],
        out_specs=pl.BlockSpec((1, D, P), lambda b: (b, 0, 0)),
        out_shape=jax.ShapeDtypeStruct((B, D, P), jnp.float32),
        scratch_shapes=[pltpu.VMEM((ED, P), jnp.float32)],
        compiler_params=pltpu.CompilerParams(
            dimension_semantics=("parallel",)),
    )(nodes, nodesT, We1, We1T, We2T, wd, be, bec, Wn1T, Wn2T, bnc)

    return outT.transpose(0, 2, 1).reshape(B, P * D)


# speedup vs baseline: 2.3424x; 2.3424x over previous
import jax
import jax.numpy as jnp
from jax.experimental import pallas as pl
from jax.experimental.pallas import tpu as pltpu

P = 256
D = 16
ED = 16
BB = 2


def _dot(a, b, dims):
    return jax.lax.dot_general(a, b, (dims, ((), ())),
                               preferred_element_type=jnp.float32)


def _gin_kernel(nodes_ref, nodesT_ref, We1_ref, We1T_ref, We2T_ref, wd_ref,
                be_ref, bec_ref, Wn1T_ref, Wn2T_ref, bnc_ref, out_ref, agg_scr):
    rows = jax.lax.broadcasted_iota(jnp.int32, (P, P), 0)
    cols = jax.lax.broadcasted_iota(jnp.int32, (P, P), 1)
    offdiag = (rows != cols).astype(jnp.float32)

    for i in range(BB):
        nodes = nodes_ref[i]
        nT = nodesT_ref[i]

        g = _dot(nT, nT, ((0,), (0,)))
        sq_row = jnp.sum(nT * nT, axis=0, keepdims=True)
        sq_col = jnp.sum(nodes * nodes, axis=1, keepdims=True)
        dist = jnp.sqrt(jnp.maximum(sq_col + sq_row - 2.0 * g, 0.0)) * offdiag

        a2 = _dot(nodes, We1_ref[...], ((1,), (0,))) + be_ref[...]
        a2T = _dot(We1T_ref[...], nT, ((1,), (0,))) + bec_ref[...]
        cT = _dot(We2T_ref[...], nT, ((1,), (0,)))

        for k in range(ED):
            m = dist * wd_ref[0, k] + a2[:, k:k + 1] + cT[k:k + 1, :]
            m = jnp.maximum(m, 0.0)
            agg_scr[k:k + 1, :] = jnp.sum(m, axis=0, keepdims=True)

        aggT = agg_scr[...] - jnp.maximum(a2T + cT, 0.0)

        newT = (_dot(Wn1T_ref[...], aggT, ((1,), (0,)))
                + _dot(Wn2T_ref[...], nT, ((1,), (0,)))
                + bnc_ref[...])
        out_ref[i] = newT


def kernel(t, h, W_e, b_e, W_n, b_n):
    del t
    B = h.shape[0]
    nodes = h.reshape(B, P, D)
    nodesT = nodes.transpose(0, 2, 1)

    We1 = W_e[:D]
    We1T = We1.T
    We2T = W_e[D:2 * D].T
    wd = W_e[2 * D:2 * D + 1]
    be = b_e.reshape(1, ED)
    bec = b_e.reshape(ED, 1)
    Wn1T = W_n[:ED].T
    Wn2T = W_n[ED:].T
    bnc = b_n.reshape(D, 1)

    full = lambda shape: pl.BlockSpec(shape, lambda b: (0,) * len(shape))
    outT = pl.pallas_call(
        _gin_kernel,
        grid=(B // BB,),
        in_specs=[
            pl.BlockSpec((BB, P, D), lambda b: (b, 0, 0)),
            pl.BlockSpec((BB, D, P), lambda b: (b, 0, 0)),
            full((D, ED)), full((ED, D)), full((ED, D)), full((1, ED)),
            full((1, ED)), full((ED, 1)), full((D, ED)), full((D, D)),
            full((D, 1)),
        ],
        out_specs=pl.BlockSpec((BB, D, P), lambda b: (b, 0, 0)),
        out_shape=jax.ShapeDtypeStruct((B, D, P), jnp.float32),
        scratch_shapes=[pltpu.VMEM((ED, P), jnp.float32)],
        compiler_params=pltpu.CompilerParams(
            dimension_semantics=("parallel",)),
    )(nodes, nodesT, We1, We1T, We2T, wd, be, bec, Wn1T, Wn2T, bnc)

    return outT.transpose(0, 2, 1).reshape(B, P * D)
